# baseline (speedup 1.0000x reference)
import functools

import jax
import jax.numpy as jnp
from jax.experimental import pallas as pl
from jax.experimental.pallas import tpu as pltpu

_H = 12
_K = 100
_BLK_Q = 1024
_BLK_N = 256


def _qkv_proj_kernel(x_ref, w_ref, b_ref, q_ref, k_ref, v_ref, *, h):
    res = (
        jnp.dot(x_ref[...], w_ref[...], preferred_element_type=jnp.float32)
        + b_ref[...]
    )
    c = x_ref.shape[1]
    hd = c // h
    for hh in range(h):
        q_ref[hh] = res[:, hh * hd:(hh + 1) * hd]
        k_ref[hh] = res[:, c + hh * hd:c + (hh + 1) * hd]
        v_ref[hh] = res[:, 2 * c + hh * hd:2 * c + (hh + 1) * hd]


def _out_proj_kernel(x_ref, w_ref, b_ref, out_ref):
    out_ref[...] = (
        jnp.dot(x_ref[...], w_ref[...], preferred_element_type=jnp.float32)
        + b_ref[...]
    )


def _key_to_f32(key):
    bits = jnp.where(key >= 0, key, key ^ jnp.int32(0x7FFFFFFF))
    return jax.lax.bitcast_convert_type(bits, jnp.float32)


def _attn_kernel(q_ref, k_ref, v_ref, out_ref, *, kk, scale):
    q = q_ref[0] * scale
    k = k_ref[0]
    v = v_ref[0]
    st = jax.lax.dot_general(
        k, q, (((1,), (1,)), ((), ())), preferred_element_type=jnp.float32
    )

    n, rows = st.shape

    lo0 = jnp.full((1, rows), -0x80000000, dtype=jnp.int32)
    hi0 = jnp.full((1, rows), 0x7FFFFFFF, dtype=jnp.int32)

    def body(_, carry):
        lo, hi = carry
        span = lo ^ hi
        mid = (lo & hi) + (span >> 1) + (span & 1)
        t = _key_to_f32(mid)
        mask = jnp.where(st >= t, jnp.float32(1), jnp.float32(0))
        cnt = jnp.sum(mask, axis=0, keepdims=True)
        ok = cnt >= kk
        return jnp.where(ok, mid, lo), jnp.where(ok, hi, mid - 1)

    thr, _ = jax.lax.fori_loop(0, 32, body, (lo0, hi0))

    sel = st >= _key_to_f32(thr)
    colmax = jnp.max(st, axis=0, keepdims=True)
    p = jnp.where(sel, jnp.exp(st - colmax), 0.0)
    denom = jnp.sum(p, axis=0, keepdims=True)
    p = p / denom
    out_ref[0] = jax.lax.dot_general(
        p, v, (((0,), (0,)), ((), ())), preferred_element_type=jnp.float32
    )


def kernel(x, qkv_w, qkv_b, proj_w, proj_b):
    b, n, c = x.shape
    h = _H
    hd = c // h
    scale = hd ** -0.5
    x2 = x.reshape(n, c)

    q, k, v = pl.pallas_call(
        functools.partial(_qkv_proj_kernel, h=h),
        grid=(n // _BLK_N,),
        in_specs=[
            pl.BlockSpec((_BLK_N, c), lambda i: (i, 0)),
            pl.BlockSpec((c, 3 * c), lambda i: (0, 0)),
            pl.BlockSpec((1, 3 * c), lambda i: (0, 0)),
        ],
        out_specs=[
            pl.BlockSpec((h, _BLK_N, hd), lambda i: (0, i, 0)),
            pl.BlockSpec((h, _BLK_N, hd), lambda i: (0, i, 0)),
            pl.BlockSpec((h, _BLK_N, hd), lambda i: (0, i, 0)),
        ],
        out_shape=[
            jax.ShapeDtypeStruct((h, n, hd), jnp.float32),
            jax.ShapeDtypeStruct((h, n, hd), jnp.float32),
            jax.ShapeDtypeStruct((h, n, hd), jnp.float32),
        ],
    )(x2, qkv_w, qkv_b.reshape(1, 3 * c))

    attn_out = pl.pallas_call(
        functools.partial(_attn_kernel, kk=_K, scale=scale),
        grid=(h, n // _BLK_Q),
        in_specs=[
            pl.BlockSpec((1, _BLK_Q, hd), lambda hh, i: (hh, i, 0)),
            pl.BlockSpec((1, n, hd), lambda hh, i: (hh, 0, 0)),
            pl.BlockSpec((1, n, hd), lambda hh, i: (hh, 0, 0)),
        ],
        out_specs=pl.BlockSpec((1, _BLK_Q, hd), lambda hh, i: (hh, i, 0)),
        out_shape=jax.ShapeDtypeStruct((h, n, hd), jnp.float32),
    )(q, k, v)

    merged = attn_out.transpose(1, 0, 2).reshape(n, c)

    out = pl.pallas_call(
        _out_proj_kernel,
        grid=(n // _BLK_N,),
        in_specs=[
            pl.BlockSpec((_BLK_N, c), lambda i: (i, 0)),
            pl.BlockSpec((c, c), lambda i: (0, 0)),
            pl.BlockSpec((1, c), lambda i: (0, 0)),
        ],
        out_specs=pl.BlockSpec((_BLK_N, c), lambda i: (i, 0)),
        out_shape=jax.ShapeDtypeStruct((n, c), jnp.float32),
    )(merged, proj_w, proj_b.reshape(1, c))

    return out.reshape(b, n, c)


# speedup vs baseline: 1.0662x; 1.0662x over previous
import functools

import jax
import jax.numpy as jnp
from jax.experimental import pallas as pl
from jax.experimental.pallas import tpu as pltpu

_H = 12
_K = 100
_BLK_Q = 512
_BLK_N = 256


def _qkv_proj_kernel(x_ref, w_ref, b_ref, q_ref, k_ref, v_ref, *, h):
    res = (
        jnp.dot(x_ref[...], w_ref[...], preferred_element_type=jnp.float32)
        + b_ref[...]
    )
    c = x_ref.shape[1]
    hd = c // h
    for hh in range(h):
        q_ref[hh] = res[:, hh * hd:(hh + 1) * hd]
        k_ref[hh] = res[:, c + hh * hd:c + (hh + 1) * hd]
        v_ref[hh] = res[:, 2 * c + hh * hd:2 * c + (hh + 1) * hd]


def _out_proj_kernel(x_ref, w_ref, b_ref, out_ref):
    out_ref[...] = (
        jnp.dot(x_ref[...], w_ref[...], preferred_element_type=jnp.float32)
        + b_ref[...]
    )


def _f32_to_key(x):
    b = jax.lax.bitcast_convert_type(x, jnp.int32)
    return jnp.where(b >= 0, b, b ^ jnp.int32(0x7FFFFFFF))


def _key_to_f32(key):
    bits = jnp.where(key >= 0, key, key ^ jnp.int32(0x7FFFFFFF))
    return jax.lax.bitcast_convert_type(bits, jnp.float32)


def _attn_kernel(q_ref, k_ref, v_ref, out_ref, *, kk, scale):
    q = q_ref[0] * scale
    k = k_ref[0]
    v = v_ref[0]
    st = jax.lax.dot_general(
        k, q, (((1,), (1,)), ((), ())), preferred_element_type=jnp.float32
    )

    n, rows = st.shape
    colmax = jnp.max(st, axis=0, keepdims=True)
    colmin = jnp.min(st, axis=0, keepdims=True)

    lo0 = _f32_to_key(colmin)
    hi0 = _f32_to_key(colmax)

    def cond(carry):
        i, lo, hi = carry
        return jnp.logical_and(i < 34, jnp.any(lo < hi))

    def body(carry):
        i, lo, hi = carry
        span = lo ^ hi
        mid = (lo & hi) + (span >> 1) + (span & 1)
        t = _key_to_f32(mid)
        mask = jnp.where(st >= t, jnp.float32(1), jnp.float32(0))
        cnt = jnp.sum(mask, axis=0, keepdims=True)
        ok = cnt >= kk
        exact = cnt == kk
        lo = jnp.where(ok, mid, lo)
        hi = jnp.where(exact, mid, jnp.where(ok, hi, mid - 1))
        return i + 1, lo, hi

    _, thr, _ = jax.lax.while_loop(cond, body, (jnp.int32(0), lo0, hi0))

    sel = st >= _key_to_f32(thr)
    p = jnp.where(sel, jnp.exp(st - colmax), 0.0)
    denom = jnp.sum(p, axis=0, keepdims=True)
    p = p / denom
    out_ref[0] = jax.lax.dot_general(
        p, v, (((0,), (0,)), ((), ())), preferred_element_type=jnp.float32
    )


def kernel(x, qkv_w, qkv_b, proj_w, proj_b):
    b, n, c = x.shape
    h = _H
    hd = c // h
    scale = hd ** -0.5
    x2 = x.reshape(n, c)

    q, k, v = pl.pallas_call(
        functools.partial(_qkv_proj_kernel, h=h),
        grid=(n // _BLK_N,),
        in_specs=[
            pl.BlockSpec((_BLK_N, c), lambda i: (i, 0)),
            pl.BlockSpec((c, 3 * c), lambda i: (0, 0)),
            pl.BlockSpec((1, 3 * c), lambda i: (0, 0)),
        ],
        out_specs=[
            pl.BlockSpec((h, _BLK_N, hd), lambda i: (0, i, 0)),
            pl.BlockSpec((h, _BLK_N, hd), lambda i: (0, i, 0)),
            pl.BlockSpec((h, _BLK_N, hd), lambda i: (0, i, 0)),
        ],
        out_shape=[
            jax.ShapeDtypeStruct((h, n, hd), jnp.float32),
            jax.ShapeDtypeStruct((h, n, hd), jnp.float32),
            jax.ShapeDtypeStruct((h, n, hd), jnp.float32),
        ],
    )(x2, qkv_w, qkv_b.reshape(1, 3 * c))

    attn_out = pl.pallas_call(
        functools.partial(_attn_kernel, kk=_K, scale=scale),
        grid=(h, n // _BLK_Q),
        in_specs=[
            pl.BlockSpec((1, _BLK_Q, hd), lambda hh, i: (hh, i, 0)),
            pl.BlockSpec((1, n, hd), lambda hh, i: (hh, 0, 0)),
            pl.BlockSpec((1, n, hd), lambda hh, i: (hh, 0, 0)),
        ],
        out_specs=pl.BlockSpec((1, _BLK_Q, hd), lambda hh, i: (hh, i, 0)),
        out_shape=jax.ShapeDtypeStruct((h, n, hd), jnp.float32),
    )(q, k, v)

    merged = attn_out.transpose(1, 0, 2).reshape(n, c)

    out = pl.pallas_call(
        _out_proj_kernel,
        grid=(n // _BLK_N,),
        in_specs=[
            pl.BlockSpec((_BLK_N, c), lambda i: (i, 0)),
            pl.BlockSpec((c, c), lambda i: (0, 0)),
            pl.BlockSpec((1, c), lambda i: (0, 0)),
        ],
        out_specs=pl.BlockSpec((_BLK_N, c), lambda i: (i, 0)),
        out_shape=jax.ShapeDtypeStruct((n, c), jnp.float32),
    )(merged, proj_w, proj_b.reshape(1, c))

    return out.reshape(b, n, c)
